# Initial kernel scaffold; baseline (speedup 1.0000x reference)
import jax
import jax.numpy as jnp
from jax.experimental import pallas as pl


def kernel(x, edge_index, batch, graph_features, W1, b1, W2, b2, Wc1, bc1, Wc2, bc2):
    raise NotImplementedError("write your pallas kernel here")


# speedup vs baseline: 27.4218x; 27.4218x over previous
import functools

import jax
import jax.numpy as jnp
from jax import lax
from jax.experimental import pallas as pl
from jax.experimental.pallas import tpu as pltpu
from jax.experimental.pallas import tpu_sc as plsc

N = 10000
E = 320000
D = 128
H = 64
G = 16
NG = 64
NC = 1

NP = 10240
NW = 32
CHUNK = 128
CHUNKS = 80
EPAD = NW * CHUNKS * CHUNK
SL = NP // 16

_mesh = plsc.VectorSubcoreMesh(core_axis_name="c", subcore_axis_name="s")
_sc_params = pltpu.CompilerParams(use_tc_tiling_on_sc=False)


@functools.partial(
    pl.kernel,
    mesh=_mesh,
    out_type=jax.ShapeDtypeStruct((2, NP), jnp.float32),
    compiler_params=_sc_params,
    scratch_types=[
        pltpu.VMEM((CHUNKS, CHUNK), jnp.int32),
        pltpu.VMEM((CHUNK,), jnp.float32),
        pltpu.VMEM((SL,), jnp.float32),
        pltpu.VMEM_SHARED((NP,), jnp.float32),
    ],
)
def _sc_degree(dst_hbm, out_hbm, didx, ones_v, zbuf, acc):
    c = lax.axis_index("c")
    s = lax.axis_index("s")
    wid = s * 2 + c
    zeros16 = jnp.zeros((16,), jnp.float32)
    ones16 = jnp.ones((16,), jnp.float32)

    def zinit(i, _):
        zbuf[pl.ds(i * 16, 16)] = zeros16
        return 0
    lax.fori_loop(0, SL // 16, zinit, 0)
    for j in range(CHUNK // 16):
        ones_v[pl.ds(j * 16, 16)] = ones16
    pltpu.sync_copy(zbuf, acc.at[pl.ds(s * SL, SL)])
    pltpu.sync_copy(dst_hbm.at[wid], didx)
    plsc.subcore_barrier()

    def body(j, _):
        pltpu.sync_copy(ones_v, acc.at[didx.at[j]], add=True)
        return 0
    lax.fori_loop(0, CHUNKS, body, 0)
    plsc.subcore_barrier()
    pltpu.sync_copy(acc.at[pl.ds(s * SL, SL)], out_hbm.at[c, pl.ds(s * SL, SL)])


@functools.partial(
    pl.kernel,
    mesh=_mesh,
    out_type=jax.ShapeDtypeStruct((2, NP, H), jnp.float32),
    compiler_params=_sc_params,
    scratch_types=[
        pltpu.VMEM((CHUNKS, CHUNK), jnp.int32),
        pltpu.VMEM((CHUNKS, CHUNK), jnp.int32),
        pltpu.VMEM((CHUNK, H), jnp.float32),
        pltpu.VMEM((64, H), jnp.float32),
        pltpu.VMEM_SHARED((NP, H), jnp.float32),
        pltpu.SemaphoreType.DMA,
    ],
)
def _sc_aggregate(xw_hbm, src_hbm, dst_hbm, out_hbm,
                  sidx, didx, rows, zbuf, acc, sem):
    c = lax.axis_index("c")
    s = lax.axis_index("s")
    wid = s * 2 + c
    zeros16 = jnp.zeros((16,), jnp.float32)

    def zinit(i, _):
        for j in range(H // 16):
            zbuf[i, pl.ds(j * 16, 16)] = zeros16
        return 0
    lax.fori_loop(0, 64, zinit, 0)

    def zcopy(t, _):
        pltpu.sync_copy(zbuf, acc.at[pl.ds(s * SL + t * 64, 64)])
        return 0
    lax.fori_loop(0, SL // 64, zcopy, 0)
    pltpu.sync_copy(src_hbm.at[wid], sidx)
    pltpu.sync_copy(dst_hbm.at[wid], didx)
    plsc.subcore_barrier()

    def body(j, _):
        pltpu.async_copy(xw_hbm.at[sidx.at[j]], rows, sem).wait()
        pltpu.sync_copy(rows, acc.at[didx.at[j]], add=True)
        return 0
    lax.fori_loop(0, CHUNKS, body, 0)
    plsc.subcore_barrier()
    pltpu.sync_copy(acc.at[pl.ds(s * SL, SL)], out_hbm.at[c, pl.ds(s * SL, SL)])


R = 1000


def _mm1_body(x_ref, w_ref, d0_ref, d1_ref, xws_ref, dinv_ref):
    deg = d0_ref[...] + d1_ref[...] + 1.0
    dinv = lax.rsqrt(deg)
    xw = jnp.dot(x_ref[...], w_ref[...], preferred_element_type=jnp.float32)
    xws_ref[...] = xw * dinv
    dinv_ref[...] = dinv


def _tc_mm1(x, W1, deg0, deg1):
    return pl.pallas_call(
        _mm1_body,
        grid=(N // R,),
        in_specs=[
            pl.BlockSpec((R, D), lambda i: (i, 0)),
            pl.BlockSpec((D, H), lambda i: (0, 0)),
            pl.BlockSpec((R, 1), lambda i: (i, 0)),
            pl.BlockSpec((R, 1), lambda i: (i, 0)),
        ],
        out_specs=[
            pl.BlockSpec((R, H), lambda i: (i, 0)),
            pl.BlockSpec((R, 1), lambda i: (i, 0)),
        ],
        out_shape=[
            jax.ShapeDtypeStruct((N, H), jnp.float32),
            jax.ShapeDtypeStruct((N, 1), jnp.float32),
        ],
    )(x, W1, deg0, deg1)


def _combine_body(a0_ref, a1_ref, xws_ref, dinv_ref, b_ref, w_ref, out_ref):
    dinv = dinv_ref[...]
    h = (a0_ref[...] + a1_ref[...] + xws_ref[...]) * dinv + b_ref[...]
    h = jnp.maximum(h, 0.0)
    out_ref[...] = jnp.dot(h, w_ref[...], preferred_element_type=jnp.float32) * dinv


def _tc_combine_mm(a0, a1, xws, dinv, b, W2):
    return pl.pallas_call(
        _combine_body,
        grid=(N // R,),
        in_specs=[
            pl.BlockSpec((R, H), lambda i: (i, 0)),
            pl.BlockSpec((R, H), lambda i: (i, 0)),
            pl.BlockSpec((R, H), lambda i: (i, 0)),
            pl.BlockSpec((R, 1), lambda i: (i, 0)),
            pl.BlockSpec((1, H), lambda i: (0, 0)),
            pl.BlockSpec((H, H), lambda i: (0, 0)),
        ],
        out_specs=pl.BlockSpec((R, H), lambda i: (i, 0)),
        out_shape=jax.ShapeDtypeStruct((N, H), jnp.float32),
    )(a0, a1, xws, dinv, b, W2)


def _final_body(a0_ref, a1_ref, xws_ref, dinv_ref, b_ref, batch_ref, gf_ref,
                wc1_ref, bc1_ref, wc2_ref, bc2_ref, out_ref, pool_acc, cnt_acc):
    i = pl.program_id(0)

    @pl.when(i == 0)
    def _():
        pool_acc[...] = jnp.zeros((NG, H), jnp.float32)
        cnt_acc[...] = jnp.zeros((NG, 1), jnp.float32)

    dinv = dinv_ref[...]
    h = (a0_ref[...] + a1_ref[...] + xws_ref[...]) * dinv + b_ref[...]
    h = jnp.maximum(h, 0.0)
    seg = lax.broadcasted_iota(jnp.int32, (R, NG), 1)
    onehot = (seg == batch_ref[...]).astype(jnp.float32)
    dn = (((0,), (0,)), ((), ()))
    pool_acc[...] += lax.dot_general(onehot, h, dn,
                                     preferred_element_type=jnp.float32)
    ones_col = jnp.ones((R, 1), jnp.float32)
    cnt_acc[...] += lax.dot_general(onehot, ones_col, dn,
                                    preferred_element_type=jnp.float32)

    @pl.when(i == pl.num_programs(0) - 1)
    def _():
        pooled = pool_acc[...] / jnp.maximum(cnt_acc[...], 1.0)
        z = jnp.concatenate([pooled, gf_ref[...]], axis=1)
        z1 = jnp.dot(z, wc1_ref[...], preferred_element_type=jnp.float32)
        z1 = jnp.maximum(z1 + bc1_ref[...], 0.0)
        out_ref[...] = jnp.dot(z1, wc2_ref[...],
                               preferred_element_type=jnp.float32) + bc2_ref[...]


def _tc_final(a0, a1, xws, dinv, b, batch2, gf, Wc1, bc1, Wc2, bc2):
    return pl.pallas_call(
        _final_body,
        grid=(N // R,),
        in_specs=[
            pl.BlockSpec((R, H), lambda i: (i, 0)),
            pl.BlockSpec((R, H), lambda i: (i, 0)),
            pl.BlockSpec((R, H), lambda i: (i, 0)),
            pl.BlockSpec((R, 1), lambda i: (i, 0)),
            pl.BlockSpec((1, H), lambda i: (0, 0)),
            pl.BlockSpec((R, 1), lambda i: (i, 0)),
            pl.BlockSpec((NG, G), lambda i: (0, 0)),
            pl.BlockSpec((H + G, H // 2), lambda i: (0, 0)),
            pl.BlockSpec((1, H // 2), lambda i: (0, 0)),
            pl.BlockSpec((H // 2, NC), lambda i: (0, 0)),
            pl.BlockSpec((1, NC), lambda i: (0, 0)),
        ],
        out_specs=pl.BlockSpec((NG, NC), lambda i: (0, 0)),
        out_shape=jax.ShapeDtypeStruct((NG, NC), jnp.float32),
        scratch_shapes=[
            pltpu.VMEM((NG, H), jnp.float32),
            pltpu.VMEM((NG, 1), jnp.float32),
        ],
    )(a0, a1, xws, dinv, b, batch2, gf, Wc1, bc1, Wc2, bc2)


def kernel(x, edge_index, batch, graph_features, W1, b1, W2, b2,
           Wc1, bc1, Wc2, bc2):
    src = edge_index[0]
    dst = edge_index[1]
    npad = EPAD - E
    pi = jnp.arange(npad, dtype=jnp.int32)
    src3 = jnp.concatenate([src, pi % N]).reshape(NW, CHUNKS, CHUNK)
    dst3 = jnp.concatenate([dst, N + pi % (NP - N)]).reshape(NW, CHUNKS, CHUNK)

    degp = _sc_degree(dst3)
    deg0 = degp[0, :N].reshape(N, 1)
    deg1 = degp[1, :N].reshape(N, 1)

    xw1s, dinv = _tc_mm1(x, W1, deg0, deg1)

    acc1 = _sc_aggregate(xw1s, src3, dst3)
    xw2s = _tc_combine_mm(acc1[0, :N], acc1[1, :N], xw1s, dinv,
                          b1.reshape(1, H), W2)

    acc2 = _sc_aggregate(xw2s, src3, dst3)
    out = _tc_final(acc2[0, :N], acc2[1, :N], xw2s, dinv,
                    b2.reshape(1, H), batch.reshape(N, 1),
                    graph_features, Wc1, bc1.reshape(1, H // 2),
                    Wc2, bc2.reshape(1, NC))
    return out


# speedup vs baseline: 43.4384x; 1.5841x over previous
import functools

import jax
import jax.numpy as jnp
from jax import lax
from jax.experimental import pallas as pl
from jax.experimental.pallas import tpu as pltpu
from jax.experimental.pallas import tpu_sc as plsc

N = 10000
E = 320000
D = 128
H = 64
G = 16
NG = 64
NC = 1

NP = 10240
NW = 32
CHUNK = 128
CHUNKS = 80
EPAD = NW * CHUNKS * CHUNK
SL = NP // 16

_mesh = plsc.VectorSubcoreMesh(core_axis_name="c", subcore_axis_name="s")
_sc_params = pltpu.CompilerParams(use_tc_tiling_on_sc=False)


@functools.partial(
    pl.kernel,
    mesh=_mesh,
    out_type=jax.ShapeDtypeStruct((2, NP), jnp.float32),
    compiler_params=_sc_params,
    scratch_types=[
        pltpu.VMEM((CHUNKS, CHUNK), jnp.int32),
        pltpu.VMEM((CHUNK,), jnp.float32),
        pltpu.VMEM((SL,), jnp.float32),
        pltpu.VMEM_SHARED((NP,), jnp.float32),
        pltpu.SemaphoreType.DMA,
        pltpu.SemaphoreType.DMA,
        pltpu.SemaphoreType.DMA,
        pltpu.SemaphoreType.DMA,
    ],
)
def _sc_degree(dst_hbm, out_hbm, didx, ones_v, zbuf, acc, s0, s1, s2, s3):
    c = lax.axis_index("c")
    s = lax.axis_index("s")
    wid = s * 2 + c
    sems = (s0, s1, s2, s3)
    zeros16 = jnp.zeros((16,), jnp.float32)
    ones16 = jnp.ones((16,), jnp.float32)

    def zinit(i, _):
        zbuf[pl.ds(i * 16, 16)] = zeros16
        return 0
    lax.fori_loop(0, SL // 16, zinit, 0)
    for j in range(CHUNK // 16):
        ones_v[pl.ds(j * 16, 16)] = ones16
    pltpu.sync_copy(zbuf, acc.at[pl.ds(s * SL, SL)])
    pltpu.sync_copy(dst_hbm.at[wid], didx)
    plsc.subcore_barrier()

    def sdesc(j, b):
        return pltpu.make_async_copy(ones_v, acc.at[didx.at[j]], sems[b])

    def body(i, _):
        for b in range(4):
            j = i * 4 + b
            sdesc(j, b).wait()
            sdesc(j + 4, b).start(add=True)
        return 0
    for b in range(4):
        sdesc(b, b).start(add=True)
    lax.fori_loop(0, CHUNKS // 4 - 1, body, 0)
    for b in range(4):
        sdesc(CHUNKS - 4 + b, b).wait()
    plsc.subcore_barrier()
    pltpu.sync_copy(acc.at[pl.ds(s * SL, SL)], out_hbm.at[c, pl.ds(s * SL, SL)])


@functools.partial(
    pl.kernel,
    mesh=_mesh,
    out_type=jax.ShapeDtypeStruct((2, NP, H), jnp.float32),
    compiler_params=_sc_params,
    scratch_types=[
        pltpu.VMEM((CHUNKS, CHUNK), jnp.int32),
        pltpu.VMEM((CHUNKS, CHUNK), jnp.int32),
        pltpu.VMEM((8, CHUNK, H), jnp.float32),
        pltpu.VMEM((64, H), jnp.float32),
        pltpu.VMEM_SHARED((NP, H), jnp.float32),
        [pltpu.SemaphoreType.DMA] * 8,
        [pltpu.SemaphoreType.DMA] * 8,
    ],
)
def _sc_aggregate(xw_hbm, src_hbm, dst_hbm, out_hbm,
                  sidx, didx, rows, zbuf, acc, gsems, ssems):
    c = lax.axis_index("c")
    s = lax.axis_index("s")
    wid = s * 2 + c
    zeros16 = jnp.zeros((16,), jnp.float32)

    def zinit(i, _):
        for j in range(H // 16):
            zbuf[i, pl.ds(j * 16, 16)] = zeros16
        return 0
    lax.fori_loop(0, 64, zinit, 0)

    def zcopy(t, _):
        pltpu.sync_copy(zbuf, acc.at[pl.ds(s * SL + t * 64, 64)])
        return 0
    lax.fori_loop(0, SL // 64, zcopy, 0)
    pltpu.sync_copy(src_hbm.at[wid], sidx)
    pltpu.sync_copy(dst_hbm.at[wid], didx)
    plsc.subcore_barrier()

    def gdesc(j, b):
        return pltpu.make_async_copy(xw_hbm.at[sidx.at[j]], rows.at[b],
                                     gsems[b])

    def sdesc(j, b):
        return pltpu.make_async_copy(rows.at[b], acc.at[didx.at[j]], ssems[b])

    GRP = 8
    T = CHUNKS // GRP
    for b in range(GRP):
        gdesc(b, b).start()

    def body(i, _):
        j0 = i * GRP
        for b in range(4):
            gdesc(j0 + b, b).wait()
            sdesc(j0 + b, b).start(add=True)
        for b in range(4):
            sdesc(j0 + b, b).wait()

            @pl.when(i < T - 1)
            def _():
                gdesc(j0 + GRP + b, b).start()
        for b in range(4, GRP):
            gdesc(j0 + b, b).wait()
            sdesc(j0 + b, b).start(add=True)
        for b in range(4, GRP):
            sdesc(j0 + b, b).wait()

            @pl.when(i < T - 1)
            def _():
                gdesc(j0 + GRP + b, b).start()
        return 0
    lax.fori_loop(0, T, body, 0)
    plsc.subcore_barrier()
    pltpu.sync_copy(acc.at[pl.ds(s * SL, SL)], out_hbm.at[c, pl.ds(s * SL, SL)])


R = 1000


def _mm1_body(x_ref, w_ref, d0_ref, d1_ref, xws_ref, dinv_ref):
    deg = d0_ref[...] + d1_ref[...] + 1.0
    dinv = lax.rsqrt(deg)
    xw = jnp.dot(x_ref[...], w_ref[...], preferred_element_type=jnp.float32)
    xws_ref[...] = xw * dinv
    dinv_ref[...] = dinv


def _tc_mm1(x, W1, deg0, deg1):
    return pl.pallas_call(
        _mm1_body,
        grid=(N // R,),
        in_specs=[
            pl.BlockSpec((R, D), lambda i: (i, 0)),
            pl.BlockSpec((D, H), lambda i: (0, 0)),
            pl.BlockSpec((R, 1), lambda i: (i, 0)),
            pl.BlockSpec((R, 1), lambda i: (i, 0)),
        ],
        out_specs=[
            pl.BlockSpec((R, H), lambda i: (i, 0)),
            pl.BlockSpec((R, 1), lambda i: (i, 0)),
        ],
        out_shape=[
            jax.ShapeDtypeStruct((N, H), jnp.float32),
            jax.ShapeDtypeStruct((N, 1), jnp.float32),
        ],
    )(x, W1, deg0, deg1)


def _combine_body(a0_ref, a1_ref, xws_ref, dinv_ref, b_ref, w_ref, out_ref):
    dinv = dinv_ref[...]
    h = (a0_ref[...] + a1_ref[...] + xws_ref[...]) * dinv + b_ref[...]
    h = jnp.maximum(h, 0.0)
    out_ref[...] = jnp.dot(h, w_ref[...], preferred_element_type=jnp.float32) * dinv


def _tc_combine_mm(a0, a1, xws, dinv, b, W2):
    return pl.pallas_call(
        _combine_body,
        grid=(N // R,),
        in_specs=[
            pl.BlockSpec((R, H), lambda i: (i, 0)),
            pl.BlockSpec((R, H), lambda i: (i, 0)),
            pl.BlockSpec((R, H), lambda i: (i, 0)),
            pl.BlockSpec((R, 1), lambda i: (i, 0)),
            pl.BlockSpec((1, H), lambda i: (0, 0)),
            pl.BlockSpec((H, H), lambda i: (0, 0)),
        ],
        out_specs=pl.BlockSpec((R, H), lambda i: (i, 0)),
        out_shape=jax.ShapeDtypeStruct((N, H), jnp.float32),
    )(a0, a1, xws, dinv, b, W2)


def _final_body(a0_ref, a1_ref, xws_ref, dinv_ref, b_ref, batch_ref, gf_ref,
                wc1_ref, bc1_ref, wc2_ref, bc2_ref, out_ref, pool_acc, cnt_acc):
    i = pl.program_id(0)

    @pl.when(i == 0)
    def _():
        pool_acc[...] = jnp.zeros((NG, H), jnp.float32)
        cnt_acc[...] = jnp.zeros((NG, 1), jnp.float32)

    dinv = dinv_ref[...]
    h = (a0_ref[...] + a1_ref[...] + xws_ref[...]) * dinv + b_ref[...]
    h = jnp.maximum(h, 0.0)
    seg = lax.broadcasted_iota(jnp.int32, (R, NG), 1)
    onehot = (seg == batch_ref[...]).astype(jnp.float32)
    dn = (((0,), (0,)), ((), ()))
    pool_acc[...] += lax.dot_general(onehot, h, dn,
                                     preferred_element_type=jnp.float32)
    ones_col = jnp.ones((R, 1), jnp.float32)
    cnt_acc[...] += lax.dot_general(onehot, ones_col, dn,
                                    preferred_element_type=jnp.float32)

    @pl.when(i == pl.num_programs(0) - 1)
    def _():
        pooled = pool_acc[...] / jnp.maximum(cnt_acc[...], 1.0)
        z = jnp.concatenate([pooled, gf_ref[...]], axis=1)
        z1 = jnp.dot(z, wc1_ref[...], preferred_element_type=jnp.float32)
        z1 = jnp.maximum(z1 + bc1_ref[...], 0.0)
        out_ref[...] = jnp.dot(z1, wc2_ref[...],
                               preferred_element_type=jnp.float32) + bc2_ref[...]


def _tc_final(a0, a1, xws, dinv, b, batch2, gf, Wc1, bc1, Wc2, bc2):
    return pl.pallas_call(
        _final_body,
        grid=(N // R,),
        in_specs=[
            pl.BlockSpec((R, H), lambda i: (i, 0)),
            pl.BlockSpec((R, H), lambda i: (i, 0)),
            pl.BlockSpec((R, H), lambda i: (i, 0)),
            pl.BlockSpec((R, 1), lambda i: (i, 0)),
            pl.BlockSpec((1, H), lambda i: (0, 0)),
            pl.BlockSpec((R, 1), lambda i: (i, 0)),
            pl.BlockSpec((NG, G), lambda i: (0, 0)),
            pl.BlockSpec((H + G, H // 2), lambda i: (0, 0)),
            pl.BlockSpec((1, H // 2), lambda i: (0, 0)),
            pl.BlockSpec((H // 2, NC), lambda i: (0, 0)),
            pl.BlockSpec((1, NC), lambda i: (0, 0)),
        ],
        out_specs=pl.BlockSpec((NG, NC), lambda i: (0, 0)),
        out_shape=jax.ShapeDtypeStruct((NG, NC), jnp.float32),
        scratch_shapes=[
            pltpu.VMEM((NG, H), jnp.float32),
            pltpu.VMEM((NG, 1), jnp.float32),
        ],
    )(a0, a1, xws, dinv, b, batch2, gf, Wc1, bc1, Wc2, bc2)


def kernel(x, edge_index, batch, graph_features, W1, b1, W2, b2,
           Wc1, bc1, Wc2, bc2):
    src = edge_index[0]
    dst = edge_index[1]
    npad = EPAD - E
    pi = jnp.arange(npad, dtype=jnp.int32)
    src3 = jnp.concatenate([src, pi % N]).reshape(NW, CHUNKS, CHUNK)
    dst3 = jnp.concatenate([dst, N + pi % (NP - N)]).reshape(NW, CHUNKS, CHUNK)

    degp = _sc_degree(dst3)
    deg0 = degp[0, :N].reshape(N, 1)
    deg1 = degp[1, :N].reshape(N, 1)

    xw1s, dinv = _tc_mm1(x, W1, deg0, deg1)

    acc1 = _sc_aggregate(xw1s, src3, dst3)
    xw2s = _tc_combine_mm(acc1[0, :N], acc1[1, :N], xw1s, dinv,
                          b1.reshape(1, H), W2)

    acc2 = _sc_aggregate(xw2s, src3, dst3)
    out = _tc_final(acc2[0, :N], acc2[1, :N], xw2s, dinv,
                    b2.reshape(1, H), batch.reshape(N, 1),
                    graph_features, Wc1, bc1.reshape(1, H // 2),
                    Wc2, bc2.reshape(1, NC))
    return out


# speedup vs baseline: 49.5797x; 1.1414x over previous
import functools

import jax
import jax.numpy as jnp
from jax import lax
from jax.experimental import pallas as pl
from jax.experimental.pallas import tpu as pltpu
from jax.experimental.pallas import tpu_sc as plsc

N = 10000
E = 320000
D = 128
H = 64
G = 16
NG = 64
NC = 1

NP = 10240
NW = 32
CHUNK = 125
CHUNKS = 80
ROWS_W = CHUNKS * CHUNK // CHUNK
SL = NP // 16
OL = N // 16

_mesh = plsc.VectorSubcoreMesh(core_axis_name="c", subcore_axis_name="s")
_sc_params = pltpu.CompilerParams(use_tc_tiling_on_sc=False)


@functools.partial(
    pl.kernel,
    mesh=_mesh,
    out_type=jax.ShapeDtypeStruct((2, NP), jnp.float32),
    compiler_params=_sc_params,
    scratch_types=[
        pltpu.VMEM((CHUNKS, CHUNK), jnp.int32),
        pltpu.VMEM((CHUNK,), jnp.float32),
        pltpu.VMEM((SL,), jnp.float32),
        pltpu.VMEM_SHARED((NP,), jnp.float32),
        [pltpu.SemaphoreType.DMA] * 4,
    ],
)
def _sc_degree(edges_hbm, out_hbm, didx, ones_v, zbuf, acc, sems):
    c = lax.axis_index("c")
    s = lax.axis_index("s")
    wid = s * 2 + c
    zeros16 = jnp.zeros((16,), jnp.float32)
    ones16 = jnp.ones((16,), jnp.float32)

    def zinit(i, _):
        zbuf[pl.ds(i * 16, 16)] = zeros16
        return 0
    lax.fori_loop(0, SL // 16, zinit, 0)
    for j in range(7):
        ones_v[pl.ds(j * 16, 16)] = ones16
    ones_v[pl.ds(CHUNK - 16, 16)] = ones16
    pltpu.sync_copy(zbuf, acc.at[pl.ds(s * SL, SL)])
    pltpu.sync_copy(edges_hbm.at[1, pl.ds(wid * CHUNKS, CHUNKS)], didx)
    plsc.subcore_barrier()

    def sdesc(j, b):
        return pltpu.make_async_copy(ones_v, acc.at[didx.at[j]], sems[b])

    def body(i, _):
        for b in range(4):
            j = i * 4 + b
            sdesc(j, b).wait()
            sdesc(j + 4, b).start(add=True)
        return 0
    for b in range(4):
        sdesc(b, b).start(add=True)
    lax.fori_loop(0, CHUNKS // 4 - 1, body, 0)
    for b in range(4):
        sdesc(CHUNKS - 4 + b, b).wait()
    plsc.subcore_barrier()
    pltpu.sync_copy(acc.at[pl.ds(s * SL, SL)], out_hbm.at[c, pl.ds(s * SL, SL)])


@functools.partial(
    pl.kernel,
    mesh=_mesh,
    out_type=jax.ShapeDtypeStruct((2, N, H), jnp.float32),
    compiler_params=_sc_params,
    scratch_types=[
        pltpu.VMEM((CHUNKS, CHUNK), jnp.int32),
        pltpu.VMEM((CHUNKS, CHUNK), jnp.int32),
        pltpu.VMEM((8, CHUNK, H), jnp.float32),
        pltpu.VMEM((64, H), jnp.float32),
        pltpu.VMEM_SHARED((NP, H), jnp.float32),
        [pltpu.SemaphoreType.DMA] * 8,
        [pltpu.SemaphoreType.DMA] * 8,
    ],
)
def _sc_aggregate(xw_hbm, edges_hbm, out_hbm,
                  sidx, didx, rows, zbuf, acc, gsems, ssems):
    c = lax.axis_index("c")
    s = lax.axis_index("s")
    wid = s * 2 + c
    zeros16 = jnp.zeros((16,), jnp.float32)

    def zinit(i, _):
        for j in range(H // 16):
            zbuf[i, pl.ds(j * 16, 16)] = zeros16
        return 0
    lax.fori_loop(0, 64, zinit, 0)

    def zcopy(t, _):
        pltpu.sync_copy(zbuf, acc.at[pl.ds(s * SL + t * 64, 64)])
        return 0
    lax.fori_loop(0, SL // 64, zcopy, 0)
    pltpu.sync_copy(edges_hbm.at[0, pl.ds(wid * CHUNKS, CHUNKS)], sidx)
    pltpu.sync_copy(edges_hbm.at[1, pl.ds(wid * CHUNKS, CHUNKS)], didx)
    plsc.subcore_barrier()

    def gdesc(j, b):
        return pltpu.make_async_copy(xw_hbm.at[sidx.at[j]], rows.at[b],
                                     gsems[b])

    def sdesc(j, b):
        return pltpu.make_async_copy(rows.at[b], acc.at[didx.at[j]], ssems[b])

    GRP = 8
    T = CHUNKS // GRP
    for b in range(GRP):
        gdesc(b, b).start()

    def body(i, _):
        j0 = i * GRP
        for b in range(4):
            gdesc(j0 + b, b).wait()
            sdesc(j0 + b, b).start(add=True)
        for b in range(4):
            sdesc(j0 + b, b).wait()

            @pl.when(i < T - 1)
            def _():
                gdesc(j0 + GRP + b, b).start()
        for b in range(4, GRP):
            gdesc(j0 + b, b).wait()
            sdesc(j0 + b, b).start(add=True)
        for b in range(4, GRP):
            sdesc(j0 + b, b).wait()

            @pl.when(i < T - 1)
            def _():
                gdesc(j0 + GRP + b, b).start()
        return 0
    lax.fori_loop(0, T, body, 0)
    plsc.subcore_barrier()
    pltpu.sync_copy(acc.at[pl.ds(s * OL, OL)], out_hbm.at[c, pl.ds(s * OL, OL)])


R = 2000


def _mm1_body(x_ref, w_ref, d0_ref, d1_ref, xws_ref, dinv_ref):
    deg = d0_ref[...] + d1_ref[...] + 1.0
    dinv = lax.rsqrt(deg)
    xw = jnp.dot(x_ref[...], w_ref[...], preferred_element_type=jnp.float32)
    xws_ref[...] = xw * dinv
    dinv_ref[...] = dinv


def _tc_mm1(x, W1, deg0, deg1):
    return pl.pallas_call(
        _mm1_body,
        grid=(N // R,),
        in_specs=[
            pl.BlockSpec((R, D), lambda i: (i, 0)),
            pl.BlockSpec((D, H), lambda i: (0, 0)),
            pl.BlockSpec((R, 1), lambda i: (i, 0)),
            pl.BlockSpec((R, 1), lambda i: (i, 0)),
        ],
        out_specs=[
            pl.BlockSpec((R, H), lambda i: (i, 0)),
            pl.BlockSpec((R, 1), lambda i: (i, 0)),
        ],
        out_shape=[
            jax.ShapeDtypeStruct((N, H), jnp.float32),
            jax.ShapeDtypeStruct((N, 1), jnp.float32),
        ],
    )(x, W1, deg0, deg1)


def _combine_body(acc_ref, xws_ref, dinv_ref, b_ref, w_ref, out_ref):
    dinv = dinv_ref[...]
    h = (acc_ref[0] + acc_ref[1] + xws_ref[...]) * dinv + b_ref[...]
    h = jnp.maximum(h, 0.0)
    out_ref[...] = jnp.dot(h, w_ref[...], preferred_element_type=jnp.float32) * dinv


def _tc_combine_mm(acc, xws, dinv, b, W2):
    return pl.pallas_call(
        _combine_body,
        grid=(N // R,),
        in_specs=[
            pl.BlockSpec((2, R, H), lambda i: (0, i, 0)),
            pl.BlockSpec((R, H), lambda i: (i, 0)),
            pl.BlockSpec((R, 1), lambda i: (i, 0)),
            pl.BlockSpec((1, H), lambda i: (0, 0)),
            pl.BlockSpec((H, H), lambda i: (0, 0)),
        ],
        out_specs=pl.BlockSpec((R, H), lambda i: (i, 0)),
        out_shape=jax.ShapeDtypeStruct((N, H), jnp.float32),
    )(acc, xws, dinv, b, W2)


def _final_body(acc_ref, xws_ref, dinv_ref, b_ref, batch_ref, gf_ref,
                wc1_ref, bc1_ref, wc2_ref, bc2_ref, out_ref, pool_acc, cnt_acc):
    i = pl.program_id(0)

    @pl.when(i == 0)
    def _():
        pool_acc[...] = jnp.zeros((NG, H), jnp.float32)
        cnt_acc[...] = jnp.zeros((NG, 1), jnp.float32)

    dinv = dinv_ref[...]
    h = (acc_ref[0] + acc_ref[1] + xws_ref[...]) * dinv + b_ref[...]
    h = jnp.maximum(h, 0.0)
    seg = lax.broadcasted_iota(jnp.int32, (R, NG), 1)
    onehot = (seg == batch_ref[...]).astype(jnp.float32)
    dn = (((0,), (0,)), ((), ()))
    pool_acc[...] += lax.dot_general(onehot, h, dn,
                                     preferred_element_type=jnp.float32)
    ones_col = jnp.ones((R, 1), jnp.float32)
    cnt_acc[...] += lax.dot_general(onehot, ones_col, dn,
                                    preferred_element_type=jnp.float32)

    @pl.when(i == pl.num_programs(0) - 1)
    def _():
        pooled = pool_acc[...] / jnp.maximum(cnt_acc[...], 1.0)
        z = jnp.concatenate([pooled, gf_ref[...]], axis=1)
        z1 = jnp.dot(z, wc1_ref[...], preferred_element_type=jnp.float32)
        z1 = jnp.maximum(z1 + bc1_ref[...], 0.0)
        out_ref[...] = jnp.dot(z1, wc2_ref[...],
                               preferred_element_type=jnp.float32) + bc2_ref[...]


def _tc_final(acc, xws, dinv, b, batch2, gf, Wc1, bc1, Wc2, bc2):
    return pl.pallas_call(
        _final_body,
        grid=(N // R,),
        in_specs=[
            pl.BlockSpec((2, R, H), lambda i: (0, i, 0)),
            pl.BlockSpec((R, H), lambda i: (i, 0)),
            pl.BlockSpec((R, 1), lambda i: (i, 0)),
            pl.BlockSpec((1, H), lambda i: (0, 0)),
            pl.BlockSpec((R, 1), lambda i: (i, 0)),
            pl.BlockSpec((NG, G), lambda i: (0, 0)),
            pl.BlockSpec((H + G, H // 2), lambda i: (0, 0)),
            pl.BlockSpec((1, H // 2), lambda i: (0, 0)),
            pl.BlockSpec((H // 2, NC), lambda i: (0, 0)),
            pl.BlockSpec((1, NC), lambda i: (0, 0)),
        ],
        out_specs=pl.BlockSpec((NG, NC), lambda i: (0, 0)),
        out_shape=jax.ShapeDtypeStruct((NG, NC), jnp.float32),
        scratch_shapes=[
            pltpu.VMEM((NG, H), jnp.float32),
            pltpu.VMEM((NG, 1), jnp.float32),
        ],
    )(acc, xws, dinv, b, batch2, gf, Wc1, bc1, Wc2, bc2)


def kernel(x, edge_index, batch, graph_features, W1, b1, W2, b2,
           Wc1, bc1, Wc2, bc2):
    edges3 = edge_index.reshape(2, E // CHUNK, CHUNK)

    degp = _sc_degree(edges3)
    deg0 = degp[0, :N].reshape(N, 1)
    deg1 = degp[1, :N].reshape(N, 1)

    xw1s, dinv = _tc_mm1(x, W1, deg0, deg1)

    acc1 = _sc_aggregate(xw1s, edges3)
    xw2s = _tc_combine_mm(acc1, xw1s, dinv, b1.reshape(1, H), W2)

    acc2 = _sc_aggregate(xw2s, edges3)
    out = _tc_final(acc2, xw2s, dinv,
                    b2.reshape(1, H), batch.reshape(N, 1),
                    graph_features, Wc1, bc1.reshape(1, H // 2),
                    Wc2, bc2.reshape(1, NC))
    return out


# speedup vs baseline: 58.3801x; 1.1775x over previous
import functools

import jax
import jax.numpy as jnp
from jax import lax
from jax.experimental import pallas as pl
from jax.experimental.pallas import tpu as pltpu
from jax.experimental.pallas import tpu_sc as plsc

N = 10000
E = 320000
D = 128
H = 64
G = 16
NG = 64
NC = 1

NP = 10240
NW = 32
CHUNK = 125
CHUNKS = 80
ROWS_W = CHUNKS * CHUNK // CHUNK
SL = NP // 16
OL = N // 16

_mesh = plsc.VectorSubcoreMesh(core_axis_name="c", subcore_axis_name="s")
_sc_params = pltpu.CompilerParams(use_tc_tiling_on_sc=False)


@functools.partial(
    pl.kernel,
    mesh=_mesh,
    out_type=jax.ShapeDtypeStruct((2, NP), jnp.float32),
    compiler_params=_sc_params,
    scratch_types=[
        pltpu.VMEM((CHUNKS, CHUNK), jnp.int32),
        pltpu.VMEM((CHUNK,), jnp.float32),
        pltpu.VMEM((SL,), jnp.float32),
        pltpu.VMEM_SHARED((NP,), jnp.float32),
        [pltpu.SemaphoreType.DMA] * 4,
    ],
)
def _sc_degree(edges_hbm, out_hbm, didx, ones_v, zbuf, acc, sems):
    c = lax.axis_index("c")
    s = lax.axis_index("s")
    wid = s * 2 + c
    zeros16 = jnp.zeros((16,), jnp.float32)
    ones16 = jnp.ones((16,), jnp.float32)

    def zinit(i, _):
        zbuf[pl.ds(i * 16, 16)] = zeros16
        return 0
    lax.fori_loop(0, SL // 16, zinit, 0)
    for j in range(7):
        ones_v[pl.ds(j * 16, 16)] = ones16
    ones_v[pl.ds(CHUNK - 16, 16)] = ones16
    pltpu.sync_copy(zbuf, acc.at[pl.ds(s * SL, SL)])
    pltpu.sync_copy(edges_hbm.at[1, pl.ds(wid * CHUNKS, CHUNKS)], didx)
    plsc.subcore_barrier()

    def sdesc(j, b):
        return pltpu.make_async_copy(ones_v, acc.at[didx.at[j]], sems[b])

    def body(i, _):
        for b in range(4):
            j = i * 4 + b
            sdesc(j, b).wait()
            sdesc(j + 4, b).start(add=True)
        return 0
    for b in range(4):
        sdesc(b, b).start(add=True)
    lax.fori_loop(0, CHUNKS // 4 - 1, body, 0)
    for b in range(4):
        sdesc(CHUNKS - 4 + b, b).wait()
    plsc.subcore_barrier()
    pltpu.sync_copy(acc.at[pl.ds(s * SL, SL)], out_hbm.at[c, pl.ds(s * SL, SL)])


@functools.partial(
    pl.kernel,
    mesh=_mesh,
    out_type=jax.ShapeDtypeStruct((2, N, H), jnp.float32),
    compiler_params=_sc_params,
    scratch_types=[
        pltpu.VMEM((CHUNKS, CHUNK), jnp.int32),
        pltpu.VMEM((CHUNKS, CHUNK), jnp.int32),
        pltpu.VMEM((8, CHUNK, H), jnp.float32),
        pltpu.VMEM((64, H), jnp.float32),
        pltpu.VMEM_SHARED((NP, H), jnp.float32),
        [pltpu.SemaphoreType.DMA] * 8,
        [pltpu.SemaphoreType.DMA] * 8,
    ],
)
def _sc_aggregate(xw_hbm, edges_hbm, out_hbm,
                  sidx, didx, rows, zbuf, acc, gsems, ssems):
    c = lax.axis_index("c")
    s = lax.axis_index("s")
    wid = s * 2 + c
    zeros16 = jnp.zeros((16,), jnp.float32)

    def zinit(i, _):
        for j in range(H // 16):
            zbuf[i, pl.ds(j * 16, 16)] = zeros16
        return 0
    lax.fori_loop(0, 64, zinit, 0)

    def zcopy(t, _):
        pltpu.sync_copy(zbuf, acc.at[pl.ds(s * SL + t * 64, 64)])
        return 0
    lax.fori_loop(0, SL // 64, zcopy, 0)
    pltpu.sync_copy(edges_hbm.at[0, pl.ds(wid * CHUNKS, CHUNKS)], sidx)
    pltpu.sync_copy(edges_hbm.at[1, pl.ds(wid * CHUNKS, CHUNKS)], didx)
    plsc.subcore_barrier()

    def gdesc(j, b):
        return pltpu.make_async_copy(xw_hbm.at[sidx.at[j]], rows.at[b],
                                     gsems[b])

    def sdesc(j, b):
        return pltpu.make_async_copy(rows.at[b], acc.at[didx.at[j]], ssems[b])

    GRP = 8
    T = CHUNKS // GRP
    for b in range(GRP):
        gdesc(b, b).start()

    def body(i, _):
        j0 = i * GRP
        for b in range(4):
            gdesc(j0 + b, b).wait()
            sdesc(j0 + b, b).start(add=True)
        for b in range(4):
            sdesc(j0 + b, b).wait()

            @pl.when(i < T - 1)
            def _():
                gdesc(j0 + GRP + b, b).start()
        for b in range(4, GRP):
            gdesc(j0 + b, b).wait()
            sdesc(j0 + b, b).start(add=True)
        for b in range(4, GRP):
            sdesc(j0 + b, b).wait()

            @pl.when(i < T - 1)
            def _():
                gdesc(j0 + GRP + b, b).start()
        return 0
    lax.fori_loop(0, T, body, 0)
    plsc.subcore_barrier()
    pltpu.sync_copy(acc.at[pl.ds(s * OL, OL)], out_hbm.at[c, pl.ds(s * OL, OL)])


R = 2000
RP = R // 2
NPK = N // 2


def _mm1_body(xt_ref, xb_ref, w_ref, dt_ref, db_ref, xws_ref, dinv_ref):
    dinv_t = lax.rsqrt(dt_ref[0] + dt_ref[1] + 1.0)
    dinv_b = lax.rsqrt(db_ref[0] + db_ref[1] + 1.0)
    dt64 = jnp.broadcast_to(dinv_t, (RP, H))
    db64 = jnp.broadcast_to(dinv_b, (RP, H))
    xwt = jnp.dot(xt_ref[...], w_ref[...], preferred_element_type=jnp.float32)
    xwb = jnp.dot(xb_ref[...], w_ref[...], preferred_element_type=jnp.float32)
    xws_ref[...] = jnp.concatenate([xwt * dt64, xwb * db64], axis=1)
    dinv_ref[...] = jnp.concatenate([dt64, db64], axis=1)


def _tc_mm1(x, W1, degT, degB):
    return pl.pallas_call(
        _mm1_body,
        grid=(N // R,),
        in_specs=[
            pl.BlockSpec((RP, D), lambda i: (i, 0)),
            pl.BlockSpec((RP, D), lambda i: (i + NPK // RP, 0)),
            pl.BlockSpec((D, H), lambda i: (0, 0)),
            pl.BlockSpec((2, RP, 1), lambda i: (0, i, 0)),
            pl.BlockSpec((2, RP, 1), lambda i: (0, i, 0)),
        ],
        out_specs=[
            pl.BlockSpec((RP, 2 * H), lambda i: (i, 0)),
            pl.BlockSpec((RP, 2 * H), lambda i: (i, 0)),
        ],
        out_shape=[
            jax.ShapeDtypeStruct((NPK, 2 * H), jnp.float32),
            jax.ShapeDtypeStruct((NPK, 2 * H), jnp.float32),
        ],
    )(x, x, W1, degT, degB)


def _combine_body(acc_ref, xws_ref, dinv_ref, b_ref, w_ref, out_ref):
    dinv = dinv_ref[...]
    h = (acc_ref[0] + acc_ref[1] + xws_ref[...]) * dinv + b_ref[...]
    h = jnp.maximum(h, 0.0)
    out_ref[...] = jnp.dot(h, w_ref[...], preferred_element_type=jnp.float32) * dinv


def _tc_combine_mm(accv, xws, dinv, b2p, W2d):
    return pl.pallas_call(
        _combine_body,
        grid=(N // R,),
        in_specs=[
            pl.BlockSpec((2, RP, 2 * H), lambda i: (0, i, 0)),
            pl.BlockSpec((RP, 2 * H), lambda i: (i, 0)),
            pl.BlockSpec((RP, 2 * H), lambda i: (i, 0)),
            pl.BlockSpec((1, 2 * H), lambda i: (0, 0)),
            pl.BlockSpec((2 * H, 2 * H), lambda i: (0, 0)),
        ],
        out_specs=pl.BlockSpec((RP, 2 * H), lambda i: (i, 0)),
        out_shape=jax.ShapeDtypeStruct((NPK, 2 * H), jnp.float32),
    )(accv, xws, dinv, b2p, W2d)


def _final_body(acc_ref, xws_ref, dinv_ref, b_ref, oht_ref, ohb_ref, gf_ref,
                wc1_ref, bc1_ref, wc2_ref, bc2_ref, out_ref, pool_acc, cnt_acc):
    i = pl.program_id(0)

    @pl.when(i == 0)
    def _():
        pool_acc[...] = jnp.zeros((NG, H), jnp.float32)
        cnt_acc[...] = jnp.zeros((NG, 1), jnp.float32)

    dinv = dinv_ref[...]
    hp = (acc_ref[0] + acc_ref[1] + xws_ref[...]) * dinv + b_ref[...]
    hp = jnp.maximum(hp, 0.0)
    ht = hp[:, :H]
    hb = hp[:, H:]
    oht = oht_ref[...]
    ohb = ohb_ref[...]
    dn = (((0,), (0,)), ((), ()))
    pool_acc[...] += (
        lax.dot_general(oht, ht, dn, preferred_element_type=jnp.float32)
        + lax.dot_general(ohb, hb, dn, preferred_element_type=jnp.float32))
    ones_col = jnp.ones((RP, 1), jnp.float32)
    cnt_acc[...] += (
        lax.dot_general(oht, ones_col, dn, preferred_element_type=jnp.float32)
        + lax.dot_general(ohb, ones_col, dn,
                          preferred_element_type=jnp.float32))

    @pl.when(i == pl.num_programs(0) - 1)
    def _():
        pooled = pool_acc[...] / jnp.maximum(cnt_acc[...], 1.0)
        z = jnp.concatenate([pooled, gf_ref[...]], axis=1)
        z1 = jnp.dot(z, wc1_ref[...], preferred_element_type=jnp.float32)
        z1 = jnp.maximum(z1 + bc1_ref[...], 0.0)
        out_ref[...] = jnp.dot(z1, wc2_ref[...],
                               preferred_element_type=jnp.float32) + bc2_ref[...]


def _tc_final(accv, xws, dinv, b2p, oht, ohb, gf, Wc1, bc1, Wc2, bc2):
    return pl.pallas_call(
        _final_body,
        grid=(N // R,),
        in_specs=[
            pl.BlockSpec((2, RP, 2 * H), lambda i: (0, i, 0)),
            pl.BlockSpec((RP, 2 * H), lambda i: (i, 0)),
            pl.BlockSpec((RP, 2 * H), lambda i: (i, 0)),
            pl.BlockSpec((1, 2 * H), lambda i: (0, 0)),
            pl.BlockSpec((RP, NG), lambda i: (i, 0)),
            pl.BlockSpec((RP, NG), lambda i: (i, 0)),
            pl.BlockSpec((NG, G), lambda i: (0, 0)),
            pl.BlockSpec((H + G, H // 2), lambda i: (0, 0)),
            pl.BlockSpec((1, H // 2), lambda i: (0, 0)),
            pl.BlockSpec((H // 2, NC), lambda i: (0, 0)),
            pl.BlockSpec((1, NC), lambda i: (0, 0)),
        ],
        out_specs=pl.BlockSpec((NG, NC), lambda i: (0, 0)),
        out_shape=jax.ShapeDtypeStruct((NG, NC), jnp.float32),
        scratch_shapes=[
            pltpu.VMEM((NG, H), jnp.float32),
            pltpu.VMEM((NG, 1), jnp.float32),
        ],
    )(accv, xws, dinv, b2p, oht, ohb, gf, Wc1, bc1, Wc2, bc2)


def kernel(x, edge_index, batch, graph_features, W1, b1, W2, b2,
           Wc1, bc1, Wc2, bc2):
    eperm = edge_index * 2 - (2 * NPK - 1) * (edge_index >= NPK)
    edges3 = eperm.reshape(2, E // CHUNK, CHUNK)

    degp = _sc_degree(edges3)
    degT = degp[:, 0:N:2, None]
    degB = degp[:, 1:N:2, None]

    xw1s_p, dinv_p = _tc_mm1(x, W1, degT, degB)

    zeros_h = jnp.zeros((H, H), jnp.float32)
    W2d = jnp.block([[W2, zeros_h], [zeros_h, W2]])
    b1p = jnp.concatenate([b1, b1]).reshape(1, 2 * H)
    b2p = jnp.concatenate([b2, b2]).reshape(1, 2 * H)

    acc1 = _sc_aggregate(xw1s_p.reshape(N, H), edges3)
    xw2s_p = _tc_combine_mm(acc1.reshape(2, NPK, 2 * H), xw1s_p,
                            dinv_p, b1p, W2d)

    acc2 = _sc_aggregate(xw2s_p.reshape(N, H), edges3)

    seg = jnp.arange(NG, dtype=batch.dtype)[None, :]
    oht = (batch[:NPK, None] == seg).astype(jnp.float32)
    ohb = (batch[NPK:, None] == seg).astype(jnp.float32)
    out = _tc_final(acc2.reshape(2, NPK, 2 * H), xw2s_p, dinv_p,
                    b2p, oht, ohb,
                    graph_features, Wc1, bc1.reshape(1, H // 2),
                    Wc2, bc2.reshape(1, NC))
    return out


# speedup vs baseline: 59.9658x; 1.0272x over previous
import functools

import jax
import jax.numpy as jnp
from jax import lax
from jax.experimental import pallas as pl
from jax.experimental.pallas import tpu as pltpu
from jax.experimental.pallas import tpu_sc as plsc

N = 10000
E = 320000
D = 128
H = 64
G = 16
NG = 64
NC = 1

NP = 10240
NW = 32
CHUNK = 125
CHUNKS = 80
ROWS_W = CHUNKS * CHUNK // CHUNK
SL = NP // 16
OL = N // 16

_mesh = plsc.VectorSubcoreMesh(core_axis_name="c", subcore_axis_name="s")
_sc_params = pltpu.CompilerParams(use_tc_tiling_on_sc=False)


@functools.partial(
    pl.kernel,
    mesh=_mesh,
    out_type=jax.ShapeDtypeStruct((2, NP), jnp.float32),
    compiler_params=_sc_params,
    scratch_types=[
        pltpu.VMEM((CHUNKS, CHUNK), jnp.int32),
        pltpu.VMEM((CHUNK,), jnp.float32),
        pltpu.VMEM((SL,), jnp.float32),
        pltpu.VMEM_SHARED((NP,), jnp.float32),
        [pltpu.SemaphoreType.DMA] * 4,
    ],
)
def _sc_degree(edges_hbm, out_hbm, didx, ones_v, zbuf, acc, sems):
    c = lax.axis_index("c")
    s = lax.axis_index("s")
    wid = s * 2 + c
    zeros16 = jnp.zeros((16,), jnp.float32)
    ones16 = jnp.ones((16,), jnp.float32)

    def zinit(i, _):
        zbuf[pl.ds(i * 16, 16)] = zeros16
        return 0
    lax.fori_loop(0, SL // 16, zinit, 0)
    for j in range(7):
        ones_v[pl.ds(j * 16, 16)] = ones16
    ones_v[pl.ds(CHUNK - 16, 16)] = ones16
    pltpu.sync_copy(zbuf, acc.at[pl.ds(s * SL, SL)])
    pltpu.sync_copy(edges_hbm.at[1, pl.ds(wid * CHUNKS, CHUNKS)], didx)
    plsc.subcore_barrier()

    def sdesc(j, b):
        return pltpu.make_async_copy(ones_v, acc.at[didx.at[j]], sems[b])

    def body(i, _):
        for b in range(4):
            j = i * 4 + b
            sdesc(j, b).wait()
            sdesc(j + 4, b).start(add=True)
        return 0
    for b in range(4):
        sdesc(b, b).start(add=True)
    lax.fori_loop(0, CHUNKS // 4 - 1, body, 0)
    for b in range(4):
        sdesc(CHUNKS - 4 + b, b).wait()
    plsc.subcore_barrier()
    pltpu.sync_copy(acc.at[pl.ds(s * SL, SL)], out_hbm.at[c, pl.ds(s * SL, SL)])


@functools.partial(
    pl.kernel,
    mesh=_mesh,
    out_type=jax.ShapeDtypeStruct((2, N, H), jnp.float32),
    compiler_params=_sc_params,
    scratch_types=[
        pltpu.VMEM((CHUNKS, CHUNK), jnp.int32),
        pltpu.VMEM((CHUNKS, CHUNK), jnp.int32),
        pltpu.VMEM((8, CHUNK, H), jnp.float32),
        pltpu.VMEM((64, H), jnp.float32),
        pltpu.VMEM_SHARED((NP, H), jnp.float32),
        [pltpu.SemaphoreType.DMA] * 8,
        [pltpu.SemaphoreType.DMA] * 8,
    ],
)
def _sc_aggregate(xw_hbm, edges_hbm, out_hbm,
                  sidx, didx, rows, zbuf, acc, gsems, ssems):
    c = lax.axis_index("c")
    s = lax.axis_index("s")
    wid = s * 2 + c
    zeros16 = jnp.zeros((16,), jnp.float32)

    def zinit(i, _):
        for j in range(H // 16):
            zbuf[i, pl.ds(j * 16, 16)] = zeros16
        return 0
    lax.fori_loop(0, 64, zinit, 0)

    def zcopy(t, _):
        pltpu.sync_copy(zbuf, acc.at[pl.ds(s * SL + t * 64, 64)])
        return 0
    lax.fori_loop(0, SL // 64, zcopy, 0)
    pltpu.sync_copy(edges_hbm.at[0, pl.ds(wid * CHUNKS, CHUNKS)], sidx)
    pltpu.sync_copy(edges_hbm.at[1, pl.ds(wid * CHUNKS, CHUNKS)], didx)
    plsc.subcore_barrier()

    def gdesc(j, b):
        return pltpu.make_async_copy(xw_hbm.at[sidx.at[j]], rows.at[b],
                                     gsems[b])

    def sdesc(j, b):
        return pltpu.make_async_copy(rows.at[b], acc.at[didx.at[j]], ssems[b])

    GRP = 8
    T = CHUNKS // GRP
    for b in range(GRP):
        gdesc(b, b).start()

    def body(i, _):
        j0 = i * GRP
        for b in range(4):
            gdesc(j0 + b, b).wait()
            sdesc(j0 + b, b).start(add=True)
        for b in range(4):
            sdesc(j0 + b, b).wait()

            @pl.when(i < T - 1)
            def _():
                gdesc(j0 + GRP + b, b).start()
        for b in range(4, GRP):
            gdesc(j0 + b, b).wait()
            sdesc(j0 + b, b).start(add=True)
        for b in range(4, GRP):
            sdesc(j0 + b, b).wait()

            @pl.when(i < T - 1)
            def _():
                gdesc(j0 + GRP + b, b).start()
        return 0
    lax.fori_loop(0, T, body, 0)
    plsc.subcore_barrier()
    pltpu.sync_copy(acc.at[pl.ds(s * OL, OL)], out_hbm.at[c, pl.ds(s * OL, OL)])


R = 2000
RP = R // 2
NPK = N // 2


def _mm1_body(xt_ref, xb_ref, w_ref, dv_ref, xws_ref, dinv_ref):
    dsum = dv_ref[0] + dv_ref[1] + 1.0
    dinv_t = lax.rsqrt(dsum[:, 0:1])
    dinv_b = lax.rsqrt(dsum[:, 1:2])
    dt64 = jnp.broadcast_to(dinv_t, (RP, H))
    db64 = jnp.broadcast_to(dinv_b, (RP, H))
    xwt = jnp.dot(xt_ref[...], w_ref[...], preferred_element_type=jnp.float32)
    xwb = jnp.dot(xb_ref[...], w_ref[...], preferred_element_type=jnp.float32)
    xws_ref[...] = jnp.concatenate([xwt * dt64, xwb * db64], axis=1)
    dinv_ref[...] = jnp.concatenate([dt64, db64], axis=1)


def _tc_mm1(x, W1, degv):
    return pl.pallas_call(
        _mm1_body,
        grid=(N // R,),
        in_specs=[
            pl.BlockSpec((RP, D), lambda i: (i, 0)),
            pl.BlockSpec((RP, D), lambda i: (i + NPK // RP, 0)),
            pl.BlockSpec((D, H), lambda i: (0, 0)),
            pl.BlockSpec((2, RP, 2), lambda i: (0, i, 0)),
        ],
        out_specs=[
            pl.BlockSpec((RP, 2 * H), lambda i: (i, 0)),
            pl.BlockSpec((RP, 2 * H), lambda i: (i, 0)),
        ],
        out_shape=[
            jax.ShapeDtypeStruct((NPK, 2 * H), jnp.float32),
            jax.ShapeDtypeStruct((NPK, 2 * H), jnp.float32),
        ],
    )(x, x, W1, degv)


def _combine_body(acc_ref, xws_ref, dinv_ref, b_ref, w_ref, out_ref):
    dinv = dinv_ref[...]
    h = (acc_ref[0] + acc_ref[1] + xws_ref[...]) * dinv + b_ref[...]
    h = jnp.maximum(h, 0.0)
    out_ref[...] = jnp.dot(h, w_ref[...], preferred_element_type=jnp.float32) * dinv


def _tc_combine_mm(accv, xws, dinv, b2p, W2d):
    return pl.pallas_call(
        _combine_body,
        grid=(N // R,),
        in_specs=[
            pl.BlockSpec((2, RP, 2 * H), lambda i: (0, i, 0)),
            pl.BlockSpec((RP, 2 * H), lambda i: (i, 0)),
            pl.BlockSpec((RP, 2 * H), lambda i: (i, 0)),
            pl.BlockSpec((1, 2 * H), lambda i: (0, 0)),
            pl.BlockSpec((2 * H, 2 * H), lambda i: (0, 0)),
        ],
        out_specs=pl.BlockSpec((RP, 2 * H), lambda i: (i, 0)),
        out_shape=jax.ShapeDtypeStruct((NPK, 2 * H), jnp.float32),
    )(accv, xws, dinv, b2p, W2d)


def _final_body(acc_ref, xws_ref, dinv_ref, b_ref, oht_ref, ohb_ref, gf_ref,
                wc1_ref, bc1_ref, wc2_ref, bc2_ref, out_ref, pool_acc, cnt_acc):
    i = pl.program_id(0)

    @pl.when(i == 0)
    def _():
        pool_acc[...] = jnp.zeros((NG, H), jnp.float32)
        cnt_acc[...] = jnp.zeros((NG, 1), jnp.float32)

    dinv = dinv_ref[...]
    hp = (acc_ref[0] + acc_ref[1] + xws_ref[...]) * dinv + b_ref[...]
    hp = jnp.maximum(hp, 0.0)
    ht = hp[:, :H]
    hb = hp[:, H:]
    oht = oht_ref[...]
    ohb = ohb_ref[...]
    dn = (((0,), (0,)), ((), ()))
    pool_acc[...] += (
        lax.dot_general(oht, ht, dn, preferred_element_type=jnp.float32)
        + lax.dot_general(ohb, hb, dn, preferred_element_type=jnp.float32))
    ones_col = jnp.ones((RP, 1), jnp.float32)
    cnt_acc[...] += (
        lax.dot_general(oht, ones_col, dn, preferred_element_type=jnp.float32)
        + lax.dot_general(ohb, ones_col, dn,
                          preferred_element_type=jnp.float32))

    @pl.when(i == pl.num_programs(0) - 1)
    def _():
        pooled = pool_acc[...] / jnp.maximum(cnt_acc[...], 1.0)
        z = jnp.concatenate([pooled, gf_ref[...]], axis=1)
        z1 = jnp.dot(z, wc1_ref[...], preferred_element_type=jnp.float32)
        z1 = jnp.maximum(z1 + bc1_ref[...], 0.0)
        out_ref[...] = jnp.dot(z1, wc2_ref[...],
                               preferred_element_type=jnp.float32) + bc2_ref[...]


def _tc_final(accv, xws, dinv, b2p, oht, ohb, gf, Wc1, bc1, Wc2, bc2):
    return pl.pallas_call(
        _final_body,
        grid=(N // R,),
        in_specs=[
            pl.BlockSpec((2, RP, 2 * H), lambda i: (0, i, 0)),
            pl.BlockSpec((RP, 2 * H), lambda i: (i, 0)),
            pl.BlockSpec((RP, 2 * H), lambda i: (i, 0)),
            pl.BlockSpec((1, 2 * H), lambda i: (0, 0)),
            pl.BlockSpec((RP, NG), lambda i: (i, 0)),
            pl.BlockSpec((RP, NG), lambda i: (i, 0)),
            pl.BlockSpec((NG, G), lambda i: (0, 0)),
            pl.BlockSpec((H + G, H // 2), lambda i: (0, 0)),
            pl.BlockSpec((1, H // 2), lambda i: (0, 0)),
            pl.BlockSpec((H // 2, NC), lambda i: (0, 0)),
            pl.BlockSpec((1, NC), lambda i: (0, 0)),
        ],
        out_specs=pl.BlockSpec((NG, NC), lambda i: (0, 0)),
        out_shape=jax.ShapeDtypeStruct((NG, NC), jnp.float32),
        scratch_shapes=[
            pltpu.VMEM((NG, H), jnp.float32),
            pltpu.VMEM((NG, 1), jnp.float32),
        ],
    )(accv, xws, dinv, b2p, oht, ohb, gf, Wc1, bc1, Wc2, bc2)


def kernel(x, edge_index, batch, graph_features, W1, b1, W2, b2,
           Wc1, bc1, Wc2, bc2):
    eperm = edge_index * 2 - (2 * NPK - 1) * (edge_index >= NPK)
    edges3 = eperm.reshape(2, E // CHUNK, CHUNK)

    degp = _sc_degree(edges3)
    degv = degp.reshape(2, NP // 2, 2)

    xw1s_p, dinv_p = _tc_mm1(x, W1, degv)

    zeros_h = jnp.zeros((H, H), jnp.float32)
    W2d = jnp.block([[W2, zeros_h], [zeros_h, W2]])
    b1p = jnp.concatenate([b1, b1]).reshape(1, 2 * H)
    b2p = jnp.concatenate([b2, b2]).reshape(1, 2 * H)

    acc1 = _sc_aggregate(xw1s_p.reshape(N, H), edges3)
    xw2s_p = _tc_combine_mm(acc1.reshape(2, NPK, 2 * H), xw1s_p,
                            dinv_p, b1p, W2d)

    acc2 = _sc_aggregate(xw2s_p.reshape(N, H), edges3)

    seg = jnp.arange(NG, dtype=batch.dtype)[None, :]
    oht = (batch[:NPK, None] == seg).astype(jnp.float32)
    ohb = (batch[NPK:, None] == seg).astype(jnp.float32)
    out = _tc_final(acc2.reshape(2, NPK, 2 * H), xw2s_p, dinv_p,
                    b2p, oht, ohb,
                    graph_features, Wc1, bc1.reshape(1, H // 2),
                    Wc2, bc2.reshape(1, NC))
    return out


# speedup vs baseline: 60.0560x; 1.0015x over previous
import functools

import jax
import jax.numpy as jnp
from jax import lax
from jax.experimental import pallas as pl
from jax.experimental.pallas import tpu as pltpu
from jax.experimental.pallas import tpu_sc as plsc

N = 10000
E = 320000
D = 128
H = 64
G = 16
NG = 64
NC = 1

NP = 10240
NW = 32
CHUNK = 125
CHUNKS = 80
ROWS_W = CHUNKS * CHUNK // CHUNK
SL = NP // 16
OL = N // 16

_mesh = plsc.VectorSubcoreMesh(core_axis_name="c", subcore_axis_name="s")
_sc_params = pltpu.CompilerParams(use_tc_tiling_on_sc=False)


@functools.partial(
    pl.kernel,
    mesh=_mesh,
    out_type=jax.ShapeDtypeStruct((2, NP), jnp.float32),
    compiler_params=_sc_params,
    scratch_types=[
        pltpu.VMEM((CHUNKS, CHUNK), jnp.int32),
        pltpu.VMEM((CHUNK,), jnp.float32),
        pltpu.VMEM((SL,), jnp.float32),
        pltpu.VMEM_SHARED((NP,), jnp.float32),
        [pltpu.SemaphoreType.DMA] * 4,
    ],
)
def _sc_degree(edges_hbm, out_hbm, didx, ones_v, zbuf, acc, sems):
    c = lax.axis_index("c")
    s = lax.axis_index("s")
    wid = s * 2 + c
    zeros16 = jnp.zeros((16,), jnp.float32)
    ones16 = jnp.ones((16,), jnp.float32)

    def zinit(i, _):
        zbuf[pl.ds(i * 16, 16)] = zeros16
        return 0
    lax.fori_loop(0, SL // 16, zinit, 0)
    for j in range(7):
        ones_v[pl.ds(j * 16, 16)] = ones16
    ones_v[pl.ds(CHUNK - 16, 16)] = ones16
    pltpu.sync_copy(zbuf, acc.at[pl.ds(s * SL, SL)])
    pltpu.sync_copy(edges_hbm.at[1, pl.ds(wid * CHUNKS, CHUNKS)], didx)
    plsc.subcore_barrier()

    def sdesc(j, b):
        return pltpu.make_async_copy(ones_v, acc.at[didx.at[j]], sems[b])

    def body(i, _):
        for b in range(4):
            j = i * 4 + b
            sdesc(j, b).wait()
            sdesc(j + 4, b).start(add=True)
        return 0
    for b in range(4):
        sdesc(b, b).start(add=True)
    lax.fori_loop(0, CHUNKS // 4 - 1, body, 0)
    for b in range(4):
        sdesc(CHUNKS - 4 + b, b).wait()
    plsc.subcore_barrier()
    pltpu.sync_copy(acc.at[pl.ds(s * SL, SL)], out_hbm.at[c, pl.ds(s * SL, SL)])


@functools.partial(
    pl.kernel,
    mesh=_mesh,
    out_type=jax.ShapeDtypeStruct((2, N, H), jnp.float32),
    compiler_params=_sc_params,
    scratch_types=[
        pltpu.VMEM((CHUNKS, CHUNK), jnp.int32),
        pltpu.VMEM((CHUNKS, CHUNK), jnp.int32),
        pltpu.VMEM((8, CHUNK, H), jnp.float32),
        pltpu.VMEM((64, H), jnp.float32),
        pltpu.VMEM_SHARED((NP, H), jnp.float32),
        [pltpu.SemaphoreType.DMA] * 8,
        [pltpu.SemaphoreType.DMA] * 8,
    ],
)
def _sc_aggregate(xw_hbm, edges_hbm, out_hbm,
                  sidx, didx, rows, zbuf, acc, gsems, ssems):
    c = lax.axis_index("c")
    s = lax.axis_index("s")
    wid = s * 2 + c
    zeros16 = jnp.zeros((16,), jnp.float32)

    def zinit(i, _):
        for j in range(H // 16):
            zbuf[i, pl.ds(j * 16, 16)] = zeros16
        return 0
    lax.fori_loop(0, 64, zinit, 0)

    def zcopy(t, _):
        pltpu.sync_copy(zbuf, acc.at[pl.ds(s * SL + t * 64, 64)])
        return 0
    lax.fori_loop(0, SL // 64, zcopy, 0)
    pltpu.sync_copy(edges_hbm.at[0, pl.ds(wid * CHUNKS, CHUNKS)], sidx)
    pltpu.sync_copy(edges_hbm.at[1, pl.ds(wid * CHUNKS, CHUNKS)], didx)
    plsc.subcore_barrier()

    def gdesc(j, b):
        return pltpu.make_async_copy(xw_hbm.at[sidx.at[j]], rows.at[b],
                                     gsems[b])

    def sdesc(j, b):
        return pltpu.make_async_copy(rows.at[b], acc.at[didx.at[j]], ssems[b])

    GRP = 8
    T = CHUNKS // GRP
    for b in range(GRP):
        gdesc(b, b).start()

    def body(i, _):
        j0 = i * GRP
        for b in range(4):
            gdesc(j0 + b, b).wait()
            sdesc(j0 + b, b).start(add=True)
        for b in range(4):
            sdesc(j0 + b, b).wait()

            @pl.when(i < T - 1)
            def _():
                gdesc(j0 + GRP + b, b).start()
        for b in range(4, GRP):
            gdesc(j0 + b, b).wait()
            sdesc(j0 + b, b).start(add=True)
        for b in range(4, GRP):
            sdesc(j0 + b, b).wait()

            @pl.when(i < T - 1)
            def _():
                gdesc(j0 + GRP + b, b).start()
        return 0
    lax.fori_loop(0, T, body, 0)
    plsc.subcore_barrier()
    pltpu.sync_copy(acc.at[pl.ds(s * OL, OL)], out_hbm.at[c, pl.ds(s * OL, OL)])


R = 2000
RP = R // 2
NPK = N // 2


def _mm1_body(xt_ref, xb_ref, w_ref, dv_ref, xws_ref, dinv_ref):
    dsum = dv_ref[0] + dv_ref[1] + 1.0
    dinv_t = lax.rsqrt(dsum[:, 0:1])
    dinv_b = lax.rsqrt(dsum[:, 1:2])
    dt64 = jnp.broadcast_to(dinv_t, (RP, H))
    db64 = jnp.broadcast_to(dinv_b, (RP, H))
    xwt = jnp.dot(xt_ref[...], w_ref[...], preferred_element_type=jnp.float32)
    xwb = jnp.dot(xb_ref[...], w_ref[...], preferred_element_type=jnp.float32)
    xws_ref[...] = jnp.concatenate([xwt * dt64, xwb * db64], axis=1)
    dinv_ref[...] = jnp.concatenate([dt64, db64], axis=1)


def _tc_mm1(x, W1, degv):
    return pl.pallas_call(
        _mm1_body,
        grid=(N // R,),
        in_specs=[
            pl.BlockSpec((RP, D), lambda i: (i, 0)),
            pl.BlockSpec((RP, D), lambda i: (i + NPK // RP, 0)),
            pl.BlockSpec((D, H), lambda i: (0, 0)),
            pl.BlockSpec((2, RP, 2), lambda i: (0, i, 0)),
        ],
        out_specs=[
            pl.BlockSpec((RP, 2 * H), lambda i: (i, 0)),
            pl.BlockSpec((RP, 2 * H), lambda i: (i, 0)),
        ],
        out_shape=[
            jax.ShapeDtypeStruct((NPK, 2 * H), jnp.float32),
            jax.ShapeDtypeStruct((NPK, 2 * H), jnp.float32),
        ],
    )(x, x, W1, degv)


def _combine_body(acc_ref, xws_ref, dinv_ref, b_ref, w_ref, out_ref):
    dinv = dinv_ref[...]
    h = (acc_ref[0] + acc_ref[1] + xws_ref[...]) * dinv + b_ref[...]
    h = jnp.maximum(h, 0.0)
    out_ref[...] = jnp.dot(h, w_ref[...], preferred_element_type=jnp.float32) * dinv


def _tc_combine_mm(accv, xws, dinv, b2p, W2d):
    return pl.pallas_call(
        _combine_body,
        grid=(N // R,),
        in_specs=[
            pl.BlockSpec((2, RP, 2 * H), lambda i: (0, i, 0)),
            pl.BlockSpec((RP, 2 * H), lambda i: (i, 0)),
            pl.BlockSpec((RP, 2 * H), lambda i: (i, 0)),
            pl.BlockSpec((1, 2 * H), lambda i: (0, 0)),
            pl.BlockSpec((2 * H, 2 * H), lambda i: (0, 0)),
        ],
        out_specs=pl.BlockSpec((RP, 2 * H), lambda i: (i, 0)),
        out_shape=jax.ShapeDtypeStruct((NPK, 2 * H), jnp.float32),
    )(accv, xws, dinv, b2p, W2d)


def _final_body(acc_ref, xws_ref, dinv_ref, b_ref, oht_ref, ohb_ref, gf_ref,
                wc1_ref, bc1_ref, wc2_ref, bc2_ref, out_ref, pool_acc, cnt_acc):
    i = pl.program_id(0)

    @pl.when(i == 0)
    def _():
        pool_acc[...] = jnp.zeros((NG, H), jnp.float32)
        cnt_acc[...] = jnp.zeros((NG, 1), jnp.float32)

    dinv = dinv_ref[...]
    hp = (acc_ref[0] + acc_ref[1] + xws_ref[...]) * dinv + b_ref[...]
    hp = jnp.maximum(hp, 0.0)
    ht = hp[:, :H]
    hb = hp[:, H:]
    oht = oht_ref[...]
    ohb = ohb_ref[...]
    dn = (((0,), (0,)), ((), ()))
    pool_acc[...] += (
        lax.dot_general(oht, ht, dn, preferred_element_type=jnp.float32)
        + lax.dot_general(ohb, hb, dn, preferred_element_type=jnp.float32))
    ones_col = jnp.ones((RP, 1), jnp.float32)
    cnt_acc[...] += (
        lax.dot_general(oht, ones_col, dn, preferred_element_type=jnp.float32)
        + lax.dot_general(ohb, ones_col, dn,
                          preferred_element_type=jnp.float32))

    @pl.when(i == pl.num_programs(0) - 1)
    def _():
        pooled = pool_acc[...] / jnp.maximum(cnt_acc[...], 1.0)
        z = jnp.concatenate([pooled, gf_ref[...]], axis=1)
        z1 = jnp.dot(z, wc1_ref[...], preferred_element_type=jnp.float32)
        z1 = jnp.maximum(z1 + bc1_ref[...], 0.0)
        out_ref[...] = jnp.dot(z1, wc2_ref[...],
                               preferred_element_type=jnp.float32) + bc2_ref[...]


def _tc_final(accv, xws, dinv, b2p, oht, ohb, gf, Wc1, bc1, Wc2, bc2):
    return pl.pallas_call(
        _final_body,
        grid=(N // R,),
        in_specs=[
            pl.BlockSpec((2, RP, 2 * H), lambda i: (0, i, 0)),
            pl.BlockSpec((RP, 2 * H), lambda i: (i, 0)),
            pl.BlockSpec((RP, 2 * H), lambda i: (i, 0)),
            pl.BlockSpec((1, 2 * H), lambda i: (0, 0)),
            pl.BlockSpec((RP, NG), lambda i: (i, 0)),
            pl.BlockSpec((RP, NG), lambda i: (i, 0)),
            pl.BlockSpec((NG, G), lambda i: (0, 0)),
            pl.BlockSpec((H + G, H // 2), lambda i: (0, 0)),
            pl.BlockSpec((1, H // 2), lambda i: (0, 0)),
            pl.BlockSpec((H // 2, NC), lambda i: (0, 0)),
            pl.BlockSpec((1, NC), lambda i: (0, 0)),
        ],
        out_specs=pl.BlockSpec((NG, NC), lambda i: (0, 0)),
        out_shape=jax.ShapeDtypeStruct((NG, NC), jnp.float32),
        scratch_shapes=[
            pltpu.VMEM((NG, H), jnp.float32),
            pltpu.VMEM((NG, 1), jnp.float32),
        ],
    )(accv, xws, dinv, b2p, oht, ohb, gf, Wc1, bc1, Wc2, bc2)


def kernel(x, edge_index, batch, graph_features, W1, b1, W2, b2,
           Wc1, bc1, Wc2, bc2):
    e3 = edge_index.reshape(2, E // CHUNK, CHUNK)
    edges3 = e3 * 2 - (2 * NPK - 1) * (e3 >= NPK)

    degp = _sc_degree(edges3)
    degv = degp.reshape(2, NP // 2, 2)

    xw1s_p, dinv_p = _tc_mm1(x, W1, degv)

    zeros_h = jnp.zeros((H, H), jnp.float32)
    W2d = jnp.block([[W2, zeros_h], [zeros_h, W2]])
    b1p = jnp.concatenate([b1, b1]).reshape(1, 2 * H)
    b2p = jnp.concatenate([b2, b2]).reshape(1, 2 * H)

    acc1 = _sc_aggregate(xw1s_p.reshape(N, H), edges3)
    xw2s_p = _tc_combine_mm(acc1.reshape(2, NPK, 2 * H), xw1s_p,
                            dinv_p, b1p, W2d)

    acc2 = _sc_aggregate(xw2s_p.reshape(N, H), edges3)

    seg = jnp.arange(NG, dtype=batch.dtype)[None, :]
    oht = (batch[:NPK, None] == seg).astype(jnp.float32)
    ohb = (batch[NPK:, None] == seg).astype(jnp.float32)
    out = _tc_final(acc2.reshape(2, NPK, 2 * H), xw2s_p, dinv_p,
                    b2p, oht, ohb,
                    graph_features, Wc1, bc1.reshape(1, H // 2),
                    Wc2, bc2.reshape(1, NC))
    return out
